# baseline (speedup 1.0000x reference)
import functools

import jax
import jax.numpy as jnp
from jax import lax
from jax.experimental import pallas as pl
from jax.experimental.pallas import tpu as pltpu
from jax.experimental.pallas import tpu_sc as plsc

_ROWS = 16384
_COLS = 512
_NW = 32
_RPW = _ROWS // _NW
_CHR = 16
_NCH = _RPW // _CHR
_VEC = 16
_CPV = _COLS // _VEC

_P5 = (2.2133659407e-05, 9.9901019572e-01, -4.8915572282e-01,
       2.8330227576e-01, -1.3011784776e-01, 3.0102226626e-02)


def _bce_vec(x, t):
    ts = t * 0.125 + 0.5
    u = jnp.exp(-jnp.abs(x))
    p = _P5[5]
    for c in (_P5[4], _P5[3], _P5[2], _P5[1], _P5[0]):
        p = p * u + c
    loss = jnp.maximum(x, 0.0) - x * ts + p
    return jnp.where(t != 0.0, loss, 0.0)


def _sc_call(p2, t2):
    mesh = plsc.VectorSubcoreMesh(core_axis_name="c", subcore_axis_name="s")

    @functools.partial(
        pl.kernel,
        mesh=mesh,
        out_type=jax.ShapeDtypeStruct((_NW, _VEC), jnp.float32),
        scratch_types=[
            pltpu.VMEM((_CHR, _COLS), jnp.float32),
            pltpu.VMEM((_CHR, _COLS), jnp.float32),
            pltpu.VMEM((_VEC,), jnp.float32),
            pltpu.SemaphoreType.DMA,
            pltpu.SemaphoreType.DMA,
        ],
    )
    def sck(p_hbm, t_hbm, out_hbm, pbuf, tbuf, accv, psem, tsem):
        wid = lax.axis_index("s") * 2 + lax.axis_index("c")
        row0 = wid * _RPW

        def chunk_body(ci, acc):
            r0 = row0 + ci * _CHR
            cp = pltpu.async_copy(p_hbm.at[pl.ds(r0, _CHR), :], pbuf, psem)
            ct = pltpu.async_copy(t_hbm.at[pl.ds(r0, _CHR), :], tbuf, tsem)
            cp.wait()
            ct.wait()

            def row_body(r, acc):
                def col_body(c, acc):
                    x = pbuf[r, pl.ds(c * _VEC, _VEC)]
                    t = tbuf[r, pl.ds(c * _VEC, _VEC)]
                    return acc + _bce_vec(x, t)

                return lax.fori_loop(0, _CPV, col_body, acc)

            return lax.fori_loop(0, _CHR, row_body, acc)

        acc = lax.fori_loop(0, _NCH, chunk_body,
                            jnp.zeros((_VEC,), jnp.float32))
        accv[...] = acc
        pltpu.sync_copy(accv, out_hbm.at[wid])

    return sck(p2, t2)


def kernel(pred_map, target_map):
    p = pred_map.reshape(_ROWS, _COLS)
    t = target_map.reshape(_ROWS, _COLS)
    parts = _sc_call(p, t)
    return jnp.sum(parts)
